# Initial kernel scaffold; baseline (speedup 1.0000x reference)
import jax
import jax.numpy as jnp
from jax.experimental import pallas as pl


def kernel(x, edge_index, edge_weight, degree, weight, bias):
    raise NotImplementedError("write your pallas kernel here")


# speedup vs baseline: 76.3095x; 76.3095x over previous
import functools

import jax
import jax.numpy as jnp
from jax import lax
from jax.experimental import pallas as pl
from jax.experimental.pallas import tpu as pltpu
from jax.experimental.pallas import tpu_sc as plsc

NC = 2
NS = 16
NW = NC * NS
L = 16
CPAD = 16
BN = 4000


def _mm_body(x_ref, w_ref, d_ref, o_ref):
    h = jnp.dot(x_ref[...], w_ref[...], preferred_element_type=jnp.float32)
    o_ref[...] = h * lax.rsqrt(d_ref[...])


def _combine_body(p_ref, d_ref, o_ref):
    o_ref[...] = (p_ref[0] + p_ref[1]) / d_ref[...]


def _final_body(q_ref, d_ref, b_ref, o_ref):
    s = (q_ref[0] + q_ref[1]) * lax.rsqrt(d_ref[...]) + b_ref[...]
    m = jnp.max(s, axis=1, keepdims=True)
    lse = jnp.log(jnp.sum(jnp.exp(s - m), axis=1, keepdims=True))
    o_ref[...] = s - m - lse


@functools.lru_cache(maxsize=None)
def _make_prop(N, E):
    R = E // 128
    CH_R = 8
    CHUNKS = R // CH_R
    base_chunks = CHUNKS // NW
    extra = CHUNKS % NW
    CH_E = CH_R * 128
    TSIZE = ((N + NS - 1) // NS + 15) // 16 * 16
    ZR = next(z for z in range(512, 7, -8) if TSIZE % z == 0)
    NZ = TSIZE // ZR

    mesh = plsc.VectorSubcoreMesh(core_axis_name="c", subcore_axis_name="s")

    @functools.partial(
        pl.kernel,
        out_type=jax.ShapeDtypeStruct((NC, N, CPAD), jnp.float32),
        mesh=mesh,
        compiler_params=pltpu.CompilerParams(use_tc_tiling_on_sc=False),
        scratch_types=[
            pltpu.VMEM_SHARED((N, CPAD), jnp.float32),
            pltpu.VMEM((CH_R, 128), jnp.int32),
            pltpu.VMEM((CH_R, 128), jnp.int32),
            pltpu.VMEM((CH_E,), jnp.float32),
            pltpu.VMEM((CH_E, CPAD), jnp.float32),
            pltpu.VMEM((ZR, CPAD), jnp.float32),
            pltpu.SemaphoreType.DMA,
        ],
    )
    def prop(h_hbm, row_hbm, col_hbm, ew_hbm, out_hbm,
             acc, row_v, col_v, ew_v, rows_v, zbuf, sem):
        c = lax.axis_index("c")
        s = lax.axis_index("s")
        wid = s * NC + c

        def zb(i, carry):
            zbuf[i] = jnp.zeros((CPAD,), jnp.float32)
            return carry
        lax.fori_loop(0, ZR, zb, 0)
        tbase = pl.multiple_of(jnp.minimum(s * TSIZE, N - TSIZE), 16)
        for t in range(NZ):
            pltpu.sync_copy(zbuf, acc.at[pl.ds(tbase + t * ZR, ZR)])
        plsc.subcore_barrier()

        nk = base_chunks + jnp.where(wid < extra, 1, 0)

        def chunk_body(k, carry):
            ck = k * NW + wid
            r0 = pl.multiple_of(ck * CH_R, CH_R)
            e0 = pl.multiple_of(ck * CH_E, CH_E)
            pltpu.sync_copy(row_hbm.at[pl.ds(r0, CH_R)], row_v)
            pltpu.sync_copy(col_hbm.at[pl.ds(r0, CH_R)], col_v)
            pltpu.sync_copy(ew_hbm.at[pl.ds(e0, CH_E)], ew_v)
            cps = [pltpu.async_copy(h_hbm.at[row_v.at[j]],
                                    rows_v.at[pl.ds(j * 128, 128)], sem)
                   for j in range(CH_R)]
            for cp in cps:
                cp.wait()

            dnums = lax.GatherDimensionNumbers(
                offset_dims=(), collapsed_slice_dims=(0,), start_index_map=(0,))

            def scale_body(g, carry2):
                b = pl.multiple_of(g * L, L)
                ew16 = ew_v[pl.ds(b, L)]
                for j in range(L):
                    ewb = lax.gather(
                        ew16, jnp.full((L, 1), j, jnp.int32), dnums, (1,),
                        mode=lax.GatherScatterMode.PROMISE_IN_BOUNDS)
                    rows_v[b + j] = rows_v[b + j] * ewb
                return carry2
            lax.fori_loop(0, CH_E // L, scale_body, 0)

            for j in range(CH_R):
                pltpu.sync_copy(rows_v.at[pl.ds(j * 128, 128)],
                                acc.at[col_v.at[j]], add=True)
            return carry
        lax.fori_loop(0, nk, chunk_body, 0)

        plsc.subcore_barrier()
        pltpu.sync_copy(acc.at[pl.ds(tbase, TSIZE)],
                        out_hbm.at[c, pl.ds(tbase, TSIZE)])

    return prop


def kernel(x, edge_index, edge_weight, degree, weight, bias):
    N, F = x.shape
    C = weight.shape[1]
    E = edge_index.shape[1]
    row = edge_index[0].reshape(E // 128, 128)
    col = edge_index[1].reshape(E // 128, 128)
    ew = edge_weight
    wpad = jnp.zeros((F, CPAD), jnp.float32).at[:, :C].set(weight)
    bpad = jnp.full((CPAD,), -1e30, jnp.float32).at[:C].set(bias).reshape(1, CPAD)
    deg = degree.reshape(N, 1)
    grid = N // BN

    h0 = pl.pallas_call(
        _mm_body,
        grid=(grid,),
        in_specs=[pl.BlockSpec((BN, F), lambda i: (i, 0)),
                  pl.BlockSpec((F, CPAD), lambda i: (0, 0)),
                  pl.BlockSpec((BN, 1), lambda i: (i, 0))],
        out_specs=pl.BlockSpec((BN, CPAD), lambda i: (i, 0)),
        out_shape=jax.ShapeDtypeStruct((N, CPAD), jnp.float32),
    )(x, wpad, deg)

    prop = _make_prop(N, E)
    p1 = prop(h0, row, col, ew)

    h1 = pl.pallas_call(
        _combine_body,
        grid=(grid,),
        in_specs=[pl.BlockSpec((NC, BN, CPAD), lambda i: (0, i, 0)),
                  pl.BlockSpec((BN, 1), lambda i: (i, 0))],
        out_specs=pl.BlockSpec((BN, CPAD), lambda i: (i, 0)),
        out_shape=jax.ShapeDtypeStruct((N, CPAD), jnp.float32),
    )(p1, deg)

    p2 = prop(h1, row, col, ew)

    out = pl.pallas_call(
        _final_body,
        grid=(grid,),
        in_specs=[pl.BlockSpec((NC, BN, CPAD), lambda i: (0, i, 0)),
                  pl.BlockSpec((BN, 1), lambda i: (i, 0)),
                  pl.BlockSpec((1, CPAD), lambda i: (0, 0))],
        out_specs=pl.BlockSpec((BN, CPAD), lambda i: (i, 0)),
        out_shape=jax.ShapeDtypeStruct((N, CPAD), jnp.float32),
    )(p2, deg, bpad)

    return out[:, :C]


# speedup vs baseline: 95.8578x; 1.2562x over previous
import functools

import jax
import jax.numpy as jnp
from jax import lax
from jax.experimental import pallas as pl
from jax.experimental.pallas import tpu as pltpu
from jax.experimental.pallas import tpu_sc as plsc

NC = 2
NS = 16
NW = NC * NS
L = 16
CPAD = 16
BN = 4000


def _mm_body(x_ref, w_ref, d_ref, o_ref):
    h = jnp.dot(x_ref[...], w_ref[...], preferred_element_type=jnp.float32)
    o_ref[...] = h * lax.rsqrt(d_ref[...])


def _combine_body(p_ref, d_ref, o_ref):
    o_ref[...] = (p_ref[0] + p_ref[1]) / d_ref[...]


def _final_body(q_ref, d_ref, b_ref, o_ref):
    s = (q_ref[0] + q_ref[1]) * lax.rsqrt(d_ref[...]) + b_ref[...]
    m = jnp.max(s, axis=1, keepdims=True)
    lse = jnp.log(jnp.sum(jnp.exp(s - m), axis=1, keepdims=True))
    o_ref[...] = s - m - lse


@functools.lru_cache(maxsize=None)
def _make_prop(N, E):
    R = E // 128
    CH_R = 8
    CHUNKS = R // CH_R
    base_chunks = CHUNKS // NW
    extra = CHUNKS % NW
    CH_E = CH_R * 128
    TSIZE = ((N + NS - 1) // NS + 15) // 16 * 16
    NZ = TSIZE // CH_E
    ZREM = TSIZE - NZ * CH_E

    mesh = plsc.VectorSubcoreMesh(core_axis_name="c", subcore_axis_name="s")

    @functools.partial(
        pl.kernel,
        out_type=jax.ShapeDtypeStruct((NC, N, CPAD), jnp.float32),
        mesh=mesh,
        compiler_params=pltpu.CompilerParams(use_tc_tiling_on_sc=False),
        scratch_types=[
            pltpu.VMEM_SHARED((N, CPAD), jnp.float32),
            pltpu.VMEM((CH_R, 128), jnp.int32),
            pltpu.VMEM((CH_R, 128), jnp.int32),
            pltpu.VMEM((CH_E,), jnp.float32),
            pltpu.VMEM((CH_E, CPAD), jnp.float32),
            pltpu.SemaphoreType.DMA,
            pltpu.SemaphoreType.DMA,
        ],
    )
    def prop(h_hbm, row_hbm, col_hbm, ew_hbm, out_hbm,
             acc, row_v, col_v, ew_v, rows_v, sg, ss):
        c = lax.axis_index("c")
        s = lax.axis_index("s")
        wid = s * NC + c

        def zb(i, carry):
            rows_v[i] = jnp.zeros((CPAD,), jnp.float32)
            return carry
        lax.fori_loop(0, CH_E, zb, 0)
        tbase = pl.multiple_of(jnp.minimum(s * TSIZE, N - TSIZE), 16)
        for t in range(NZ):
            pltpu.sync_copy(rows_v, acc.at[pl.ds(tbase + t * CH_E, CH_E)])
        if ZREM:
            pltpu.sync_copy(rows_v.at[pl.ds(0, ZREM)],
                            acc.at[pl.ds(tbase + NZ * CH_E, ZREM)])
        plsc.subcore_barrier()

        nk = base_chunks + jnp.where(wid < extra, 1, 0)
        dnums = lax.GatherDimensionNumbers(
            offset_dims=(), collapsed_slice_dims=(0,), start_index_map=(0,))

        def sub(j):
            return rows_v.at[pl.ds(j * 128, 128)]

        def chunk_body(k, carry):
            ck = k * NW + wid
            r0 = pl.multiple_of(ck * CH_R, CH_R)
            e0 = pl.multiple_of(ck * CH_E, CH_E)
            @pl.when(k > 0)
            def _():
                for j in range(CH_R):
                    pltpu.make_async_copy(sub(j), acc.at[col_v.at[j]],
                                          ss).wait()
            pltpu.sync_copy(row_hbm.at[pl.ds(r0, CH_R)], row_v)
            pltpu.sync_copy(col_hbm.at[pl.ds(r0, CH_R)], col_v)
            pltpu.sync_copy(ew_hbm.at[pl.ds(e0, CH_E)], ew_v)
            gath = [pltpu.async_copy(h_hbm.at[row_v.at[j]], sub(j), sg)
                    for j in range(CH_R)]
            for j in range(CH_R):
                gath[j].wait()

                def scale_body(g, carry2, j=j):
                    o = pl.multiple_of(j * 128 + g * L, L)
                    ew16 = ew_v[pl.ds(o, L)]
                    for t in range(L):
                        ewb = lax.gather(
                            ew16, jnp.full((L, 1), t, jnp.int32), dnums, (1,),
                            mode=lax.GatherScatterMode.PROMISE_IN_BOUNDS)
                        rows_v[o + t] = rows_v[o + t] * ewb
                    return carry2
                lax.fori_loop(0, 128 // L, scale_body, 0)
                pltpu.async_copy(sub(j), acc.at[col_v.at[j]], ss, add=True)
            return carry
        lax.fori_loop(0, nk, chunk_body, 0)

        for j in range(CH_R):
            pltpu.make_async_copy(sub(j), acc.at[col_v.at[j]], ss).wait()

        plsc.subcore_barrier()
        pltpu.sync_copy(acc.at[pl.ds(tbase, TSIZE)],
                        out_hbm.at[c, pl.ds(tbase, TSIZE)])

    return prop


def kernel(x, edge_index, edge_weight, degree, weight, bias):
    N, F = x.shape
    C = weight.shape[1]
    E = edge_index.shape[1]
    row = edge_index[0].reshape(E // 128, 128)
    col = edge_index[1].reshape(E // 128, 128)
    ew = edge_weight
    wpad = jnp.zeros((F, CPAD), jnp.float32).at[:, :C].set(weight)
    bpad = jnp.full((CPAD,), -1e30, jnp.float32).at[:C].set(bias).reshape(1, CPAD)
    deg = degree.reshape(N, 1)
    grid = N // BN

    h0 = pl.pallas_call(
        _mm_body,
        grid=(grid,),
        in_specs=[pl.BlockSpec((BN, F), lambda i: (i, 0)),
                  pl.BlockSpec((F, CPAD), lambda i: (0, 0)),
                  pl.BlockSpec((BN, 1), lambda i: (i, 0))],
        out_specs=pl.BlockSpec((BN, CPAD), lambda i: (i, 0)),
        out_shape=jax.ShapeDtypeStruct((N, CPAD), jnp.float32),
    )(x, wpad, deg)

    prop = _make_prop(N, E)
    p1 = prop(h0, row, col, ew)

    h1 = pl.pallas_call(
        _combine_body,
        grid=(grid,),
        in_specs=[pl.BlockSpec((NC, BN, CPAD), lambda i: (0, i, 0)),
                  pl.BlockSpec((BN, 1), lambda i: (i, 0))],
        out_specs=pl.BlockSpec((BN, CPAD), lambda i: (i, 0)),
        out_shape=jax.ShapeDtypeStruct((N, CPAD), jnp.float32),
    )(p1, deg)

    p2 = prop(h1, row, col, ew)

    out = pl.pallas_call(
        _final_body,
        grid=(grid,),
        in_specs=[pl.BlockSpec((NC, BN, CPAD), lambda i: (0, i, 0)),
                  pl.BlockSpec((BN, 1), lambda i: (i, 0)),
                  pl.BlockSpec((1, CPAD), lambda i: (0, 0))],
        out_specs=pl.BlockSpec((BN, CPAD), lambda i: (i, 0)),
        out_shape=jax.ShapeDtypeStruct((N, CPAD), jnp.float32),
    )(p2, deg, bpad)

    return out[:, :C]


# speedup vs baseline: 138.2444x; 1.4422x over previous
import functools

import jax
import jax.numpy as jnp
from jax import lax
from jax.experimental import pallas as pl
from jax.experimental.pallas import tpu as pltpu
from jax.experimental.pallas import tpu_sc as plsc

NC = 2
NS = 16
NW = NC * NS
L = 16
CPAD = 16
BN = 4000


def _mm_body(x_ref, w_ref, d_ref, o_ref):
    h = jnp.dot(x_ref[...], w_ref[...], preferred_element_type=jnp.float32)
    o_ref[...] = h * lax.rsqrt(d_ref[...])


def _combine_body(p_ref, d_ref, o_ref):
    o_ref[...] = (p_ref[0] + p_ref[1]) / d_ref[...]


def _final_body(q_ref, d_ref, b_ref, o_ref):
    s = (q_ref[0] + q_ref[1]) * lax.rsqrt(d_ref[...]) + b_ref[...]
    m = jnp.max(s, axis=1, keepdims=True)
    lse = jnp.log(jnp.sum(jnp.exp(s - m), axis=1, keepdims=True))
    o_ref[...] = s - m - lse


@functools.lru_cache(maxsize=None)
def _make_prop(N, E):
    R = E // 128
    CH_R = 8
    CHUNKS = R // CH_R
    base_chunks = CHUNKS // NW
    extra = CHUNKS % NW
    CH_E = CH_R * 128
    TSIZE = ((N + NS - 1) // NS + 15) // 16 * 16
    NZ = TSIZE // CH_E
    ZREM = TSIZE - NZ * CH_E

    mesh = plsc.VectorSubcoreMesh(core_axis_name="c", subcore_axis_name="s")

    @functools.partial(
        pl.kernel,
        out_type=jax.ShapeDtypeStruct((NC, N, CPAD), jnp.float32),
        mesh=mesh,
        compiler_params=pltpu.CompilerParams(use_tc_tiling_on_sc=False),
        scratch_types=[
            pltpu.VMEM_SHARED((N, CPAD), jnp.float32),
            pltpu.VMEM((2, CH_R, 128), jnp.int32),
            pltpu.VMEM((2, CH_R, 128), jnp.int32),
            pltpu.VMEM((2, CH_E), jnp.float32),
            pltpu.VMEM((CH_E, CPAD), jnp.float32),
            pltpu.SemaphoreType.DMA,
            pltpu.SemaphoreType.DMA,
            pltpu.SemaphoreType.DMA,
            pltpu.SemaphoreType.DMA,
        ],
    )
    def prop(h_hbm, row_hbm, col_hbm, ew_hbm, out_hbm,
             acc, row_v, col_v, ew_v, rows_v, sg, ss, sl0, sl1):
        c = lax.axis_index("c")
        s = lax.axis_index("s")
        wid = s * NC + c

        def zb(i, carry):
            rows_v[i] = jnp.zeros((CPAD,), jnp.float32)
            return carry
        lax.fori_loop(0, CH_E, zb, 0)
        tbase = pl.multiple_of(jnp.minimum(s * TSIZE, N - TSIZE), 16)
        for t in range(NZ):
            pltpu.sync_copy(rows_v, acc.at[pl.ds(tbase + t * CH_E, CH_E)])
        if ZREM:
            pltpu.sync_copy(rows_v.at[pl.ds(0, ZREM)],
                            acc.at[pl.ds(tbase + NZ * CH_E, ZREM)])
        plsc.subcore_barrier()

        nk = base_chunks + jnp.where(wid < extra, 1, 0)
        dnums = lax.GatherDimensionNumbers(
            offset_dims=(), collapsed_slice_dims=(0,), start_index_map=(0,))
        sls = (sl0, sl1)

        def sub(j):
            return rows_v.at[pl.ds(j * 128, 128)]

        def lin_refs(k, b):
            ck = k * NW + wid
            r0 = pl.multiple_of(ck * CH_R, CH_R)
            e0 = pl.multiple_of(ck * CH_E, CH_E)
            return ((row_hbm.at[pl.ds(r0, CH_R)], row_v.at[b], sls[b]),
                    (col_hbm.at[pl.ds(r0, CH_R)], col_v.at[b], sls[b]),
                    (ew_hbm.at[pl.ds(e0, CH_E)], ew_v.at[b], sls[b]))

        def fire_lin(k, b):
            for src, dst, sem in lin_refs(k, b):
                pltpu.async_copy(src, dst, sem)

        def wait_lin(k, b):
            for src, dst, sem in lin_refs(k, b):
                pltpu.make_async_copy(src, dst, sem).wait()

        def drain_scatters(b):
            for j in range(CH_R):
                pltpu.make_async_copy(sub(j), acc.at[col_v.at[b].at[j]],
                                      ss).wait()

        def phase(k, b):
            @pl.when(k < nk)
            def _():
                @pl.when(k > 0)
                def _():
                    drain_scatters(1 - b)
                wait_lin(k, b)
                gath = [pltpu.async_copy(h_hbm.at[row_v.at[b].at[j]],
                                         sub(j), sg)
                        for j in range(CH_R)]

                @pl.when(k + 1 < nk)
                def _():
                    fire_lin(k + 1, 1 - b)
                for j in range(CH_R):
                    gath[j].wait()

                    def scale_body(g, carry2, j=j):
                        o = pl.multiple_of(j * 128 + g * L, L)
                        ew16 = ew_v.at[b][pl.ds(o, L)]
                        for t in range(L):
                            ewb = lax.gather(
                                ew16, jnp.full((L, 1), t, jnp.int32), dnums,
                                (1,),
                                mode=lax.GatherScatterMode.PROMISE_IN_BOUNDS)
                            rows_v[o + t] = rows_v[o + t] * ewb
                        return carry2
                    lax.fori_loop(0, 128 // L, scale_body, 0)
                    pltpu.async_copy(sub(j), acc.at[col_v.at[b].at[j]],
                                     ss, add=True)

        fire_lin(0, 0)

        def pair_body(p, carry):
            phase(p * 2, 0)
            phase(p * 2 + 1, 1)
            return carry
        lax.fori_loop(0, (base_chunks + 2) // 2, pair_body, 0)

        drain_scatters(0)

        plsc.subcore_barrier()
        pltpu.sync_copy(acc.at[pl.ds(tbase, TSIZE)],
                        out_hbm.at[c, pl.ds(tbase, TSIZE)])

    return prop


def kernel(x, edge_index, edge_weight, degree, weight, bias):
    N, F = x.shape
    C = weight.shape[1]
    E = edge_index.shape[1]
    row = edge_index[0].reshape(E // 128, 128)
    col = edge_index[1].reshape(E // 128, 128)
    ew = edge_weight
    wpad = jnp.zeros((F, CPAD), jnp.float32).at[:, :C].set(weight)
    bpad = jnp.full((CPAD,), -1e30, jnp.float32).at[:C].set(bias).reshape(1, CPAD)
    deg = degree.reshape(N, 1)
    grid = N // BN

    h0 = pl.pallas_call(
        _mm_body,
        grid=(grid,),
        in_specs=[pl.BlockSpec((BN, F), lambda i: (i, 0)),
                  pl.BlockSpec((F, CPAD), lambda i: (0, 0)),
                  pl.BlockSpec((BN, 1), lambda i: (i, 0))],
        out_specs=pl.BlockSpec((BN, CPAD), lambda i: (i, 0)),
        out_shape=jax.ShapeDtypeStruct((N, CPAD), jnp.float32),
    )(x, wpad, deg)

    prop = _make_prop(N, E)
    p1 = prop(h0, row, col, ew)

    h1 = pl.pallas_call(
        _combine_body,
        grid=(grid,),
        in_specs=[pl.BlockSpec((NC, BN, CPAD), lambda i: (0, i, 0)),
                  pl.BlockSpec((BN, 1), lambda i: (i, 0))],
        out_specs=pl.BlockSpec((BN, CPAD), lambda i: (i, 0)),
        out_shape=jax.ShapeDtypeStruct((N, CPAD), jnp.float32),
    )(p1, deg)

    p2 = prop(h1, row, col, ew)

    out = pl.pallas_call(
        _final_body,
        grid=(grid,),
        in_specs=[pl.BlockSpec((NC, BN, CPAD), lambda i: (0, i, 0)),
                  pl.BlockSpec((BN, 1), lambda i: (i, 0)),
                  pl.BlockSpec((1, CPAD), lambda i: (0, 0))],
        out_specs=pl.BlockSpec((BN, CPAD), lambda i: (i, 0)),
        out_shape=jax.ShapeDtypeStruct((N, CPAD), jnp.float32),
    )(p2, deg, bpad)

    return out[:, :C]


# speedup vs baseline: 151.9650x; 1.0992x over previous
import functools

import jax
import jax.numpy as jnp
from jax import lax
from jax.experimental import pallas as pl
from jax.experimental.pallas import tpu as pltpu
from jax.experimental.pallas import tpu_sc as plsc

NC = 2
NS = 16
NW = NC * NS
L = 16
CPAD = 16
BN = 4000


def _mm_body(x_ref, w_ref, d_ref, o_ref):
    h = jnp.dot(x_ref[...], w_ref[...], preferred_element_type=jnp.float32)
    o_ref[...] = h * lax.rsqrt(d_ref[...])


def _combine_body(p_ref, dp_ref, o_ref):
    o_ref[...] = (p_ref[0] + p_ref[1]) / dp_ref[...]


def _final_body(q_ref, d_ref, b_ref, o_ref):
    s = (q_ref[0] + q_ref[1]) * lax.rsqrt(d_ref[...]) + b_ref[...]
    m = jnp.max(s, axis=1, keepdims=True)
    lse = jnp.log(jnp.sum(jnp.exp(s - m), axis=1, keepdims=True))
    o_ref[...] = s - m - lse


@functools.lru_cache(maxsize=None)
def _make_prop(N, E):
    R = E // 128
    CH_R = 8
    CHUNKS = R // CH_R
    base_chunks = CHUNKS // NW
    extra = CHUNKS % NW
    CH_E = CH_R * 128
    TSIZE = ((N + NS - 1) // NS + 15) // 16 * 16
    NZ = TSIZE // CH_E
    ZREM = TSIZE - NZ * CH_E

    mesh = plsc.VectorSubcoreMesh(core_axis_name="c", subcore_axis_name="s")

    @functools.partial(
        pl.kernel,
        out_type=jax.ShapeDtypeStruct((NC, N, CPAD), jnp.float32),
        mesh=mesh,
        compiler_params=pltpu.CompilerParams(use_tc_tiling_on_sc=False),
        scratch_types=[
            pltpu.VMEM_SHARED((N, CPAD), jnp.float32),
            pltpu.VMEM((2, CH_R, 128), jnp.int32),
            pltpu.VMEM((2, CH_R, 128), jnp.int32),
            pltpu.VMEM((2, CH_E), jnp.float32),
            pltpu.VMEM((CH_E, CPAD), jnp.float32),
            pltpu.SemaphoreType.DMA,
            pltpu.SemaphoreType.DMA,
            pltpu.SemaphoreType.DMA,
            pltpu.SemaphoreType.DMA,
        ],
    )
    def prop(h_hbm, row_hbm, col_hbm, ew_hbm, out_hbm,
             acc, row_v, col_v, ew_v, rows_v, sg, ss, sl0, sl1):
        c = lax.axis_index("c")
        s = lax.axis_index("s")
        wid = s * NC + c

        def zb(i, carry):
            rows_v[i] = jnp.zeros((CPAD,), jnp.float32)
            return carry
        lax.fori_loop(0, CH_E, zb, 0)
        tbase = pl.multiple_of(jnp.minimum(s * TSIZE, N - TSIZE), 16)
        for t in range(NZ):
            pltpu.sync_copy(rows_v, acc.at[pl.ds(tbase + t * CH_E, CH_E)])
        if ZREM:
            pltpu.sync_copy(rows_v.at[pl.ds(0, ZREM)],
                            acc.at[pl.ds(tbase + NZ * CH_E, ZREM)])
        plsc.subcore_barrier()

        nk = base_chunks + jnp.where(wid < extra, 1, 0)
        dnums = lax.GatherDimensionNumbers(
            offset_dims=(), collapsed_slice_dims=(0,), start_index_map=(0,))
        sls = (sl0, sl1)

        def sub(j):
            return rows_v.at[pl.ds(j * 128, 128)]

        def lin_refs(k, b):
            ck = k * NW + wid
            r0 = pl.multiple_of(ck * CH_R, CH_R)
            e0 = pl.multiple_of(ck * CH_E, CH_E)
            return ((row_hbm.at[pl.ds(r0, CH_R)], row_v.at[b], sls[b]),
                    (col_hbm.at[pl.ds(r0, CH_R)], col_v.at[b], sls[b]),
                    (ew_hbm.at[pl.ds(e0, CH_E)], ew_v.at[b], sls[b]))

        def fire_lin(k, b):
            for src, dst, sem in lin_refs(k, b):
                pltpu.async_copy(src, dst, sem)

        def wait_lin(k, b):
            for src, dst, sem in lin_refs(k, b):
                pltpu.make_async_copy(src, dst, sem).wait()

        def drain_scatters(b):
            for j in range(CH_R):
                pltpu.make_async_copy(sub(j), acc.at[col_v.at[b].at[j]],
                                      ss).wait()

        def phase(k, b):
            @pl.when(k < nk)
            def _():
                @pl.when(k > 0)
                def _():
                    drain_scatters(1 - b)
                wait_lin(k, b)
                gath = [pltpu.async_copy(h_hbm.at[row_v.at[b].at[j]],
                                         sub(j), sg)
                        for j in range(CH_R)]

                @pl.when(k + 1 < nk)
                def _():
                    fire_lin(k + 1, 1 - b)
                for j in range(CH_R):
                    gath[j].wait()

                    def scale_body(g, carry2, j=j):
                        o = pl.multiple_of(j * 128 + g * L, L)
                        ew16 = ew_v.at[b][pl.ds(o, L)]
                        for t in range(L):
                            ewb = lax.gather(
                                ew16, jnp.full((L, 1), t, jnp.int32), dnums,
                                (1,),
                                mode=lax.GatherScatterMode.PROMISE_IN_BOUNDS)
                            rows_v[o + t] = rows_v[o + t] * ewb
                        return carry2
                    lax.fori_loop(0, 128 // L, scale_body, 0)
                    pltpu.async_copy(sub(j), acc.at[col_v.at[b].at[j]],
                                     ss, add=True)

        fire_lin(0, 0)

        def pair_body(p, carry):
            phase(p * 2, 0)
            phase(p * 2 + 1, 1)
            return carry
        lax.fori_loop(0, (base_chunks + 2) // 2, pair_body, 0)

        drain_scatters(0)

        plsc.subcore_barrier()
        pltpu.sync_copy(acc.at[pl.ds(tbase, TSIZE)],
                        out_hbm.at[c, pl.ds(tbase, TSIZE)])

    return prop


def kernel(x, edge_index, edge_weight, degree, weight, bias):
    N, F = x.shape
    C = weight.shape[1]
    E = edge_index.shape[1]
    row = edge_index[0].reshape(E // 128, 128)
    col = edge_index[1].reshape(E // 128, 128)
    ew = edge_weight
    wpad = jnp.zeros((F, CPAD), jnp.float32).at[:, :C].set(weight)
    bpad = jnp.full((CPAD,), -1e30, jnp.float32).at[:C].set(bias).reshape(1, CPAD)
    deg = degree.reshape(N, 1)
    NP = N * CPAD // 128
    degp = jnp.repeat(degree, CPAD).reshape(NP, 128)
    grid = N // BN
    BNP = BN * CPAD // 128

    h0 = pl.pallas_call(
        _mm_body,
        grid=(grid,),
        in_specs=[pl.BlockSpec((BN, F), lambda i: (i, 0)),
                  pl.BlockSpec((F, CPAD), lambda i: (0, 0)),
                  pl.BlockSpec((BN, 1), lambda i: (i, 0))],
        out_specs=pl.BlockSpec((BN, CPAD), lambda i: (i, 0)),
        out_shape=jax.ShapeDtypeStruct((N, CPAD), jnp.float32),
    )(x, wpad, deg)

    prop = _make_prop(N, E)
    p1 = prop(h0, row, col, ew)

    h1p = pl.pallas_call(
        _combine_body,
        in_specs=[pl.BlockSpec((NC, NP, 128), lambda: (0, 0, 0)),
                  pl.BlockSpec((NP, 128), lambda: (0, 0))],
        out_specs=pl.BlockSpec((NP, 128), lambda: (0, 0)),
        out_shape=jax.ShapeDtypeStruct((NP, 128), jnp.float32),
    )(p1.reshape(NC, NP, 128), degp)

    p2 = prop(h1p.reshape(N, CPAD), row, col, ew)

    out = pl.pallas_call(
        _final_body,
        grid=(grid,),
        in_specs=[pl.BlockSpec((NC, BN, CPAD), lambda i: (0, i, 0)),
                  pl.BlockSpec((BN, 1), lambda i: (i, 0)),
                  pl.BlockSpec((1, CPAD), lambda i: (0, 0))],
        out_specs=pl.BlockSpec((BN, CPAD), lambda i: (i, 0)),
        out_shape=jax.ShapeDtypeStruct((N, CPAD), jnp.float32),
    )(p2, deg, bpad)

    return out[:, :C]


# speedup vs baseline: 151.9804x; 1.0001x over previous
import functools

import jax
import jax.numpy as jnp
from jax import lax
from jax.experimental import pallas as pl
from jax.experimental.pallas import tpu as pltpu
from jax.experimental.pallas import tpu_sc as plsc

NC = 2
NS = 16
NW = NC * NS
L = 16
CPAD = 16
BN = 4000


def _mm_body(x_ref, w_ref, d_ref, o_ref):
    h = jnp.dot(x_ref[...], w_ref[...], preferred_element_type=jnp.float32)
    o_ref[...] = h * lax.rsqrt(d_ref[...])


def _combine_body(p_ref, dp_ref, o_ref):
    o_ref[...] = (p_ref[0] + p_ref[1]) / dp_ref[...]


def _final_body(q_ref, d_ref, b_ref, o_ref):
    s = (q_ref[0] + q_ref[1]) * lax.rsqrt(d_ref[...]) + b_ref[...]
    m = jnp.max(s, axis=1, keepdims=True)
    lse = jnp.log(jnp.sum(jnp.exp(s - m), axis=1, keepdims=True))
    r = s - m - lse
    o_ref[...] = r[:, :o_ref.shape[1]]


@functools.lru_cache(maxsize=None)
def _make_prop(N, E):
    R = E // 128
    CH_R = 8
    CHUNKS = R // CH_R
    base_chunks = CHUNKS // NW
    extra = CHUNKS % NW
    CH_E = CH_R * 128
    TSIZE = ((N + NS - 1) // NS + 15) // 16 * 16
    NZ = TSIZE // CH_E
    ZREM = TSIZE - NZ * CH_E

    mesh = plsc.VectorSubcoreMesh(core_axis_name="c", subcore_axis_name="s")

    @functools.partial(
        pl.kernel,
        out_type=jax.ShapeDtypeStruct((NC, N, CPAD), jnp.float32),
        mesh=mesh,
        compiler_params=pltpu.CompilerParams(use_tc_tiling_on_sc=False),
        scratch_types=[
            pltpu.VMEM_SHARED((N, CPAD), jnp.float32),
            pltpu.VMEM((2, CH_R, 128), jnp.int32),
            pltpu.VMEM((2, CH_R, 128), jnp.int32),
            pltpu.VMEM((2, CH_E), jnp.float32),
            pltpu.VMEM((CH_E, CPAD), jnp.float32),
            pltpu.SemaphoreType.DMA,
            pltpu.SemaphoreType.DMA,
            pltpu.SemaphoreType.DMA,
            pltpu.SemaphoreType.DMA,
        ],
    )
    def prop(h_hbm, row_hbm, col_hbm, ew_hbm, out_hbm,
             acc, row_v, col_v, ew_v, rows_v, sg, ss, sl0, sl1):
        c = lax.axis_index("c")
        s = lax.axis_index("s")
        wid = s * NC + c

        def zb(i, carry):
            rows_v[i] = jnp.zeros((CPAD,), jnp.float32)
            return carry
        lax.fori_loop(0, CH_E, zb, 0)
        tbase = pl.multiple_of(jnp.minimum(s * TSIZE, N - TSIZE), 16)
        for t in range(NZ):
            pltpu.sync_copy(rows_v, acc.at[pl.ds(tbase + t * CH_E, CH_E)])
        if ZREM:
            pltpu.sync_copy(rows_v.at[pl.ds(0, ZREM)],
                            acc.at[pl.ds(tbase + NZ * CH_E, ZREM)])
        plsc.subcore_barrier()

        nk = base_chunks + jnp.where(wid < extra, 1, 0)
        dnums = lax.GatherDimensionNumbers(
            offset_dims=(), collapsed_slice_dims=(0,), start_index_map=(0,))
        sls = (sl0, sl1)

        def sub(j):
            return rows_v.at[pl.ds(j * 128, 128)]

        def lin_refs(k, b):
            ck = k * NW + wid
            r0 = pl.multiple_of(ck * CH_R, CH_R)
            e0 = pl.multiple_of(ck * CH_E, CH_E)
            return ((row_hbm.at[pl.ds(r0, CH_R)], row_v.at[b], sls[b]),
                    (col_hbm.at[pl.ds(r0, CH_R)], col_v.at[b], sls[b]),
                    (ew_hbm.at[pl.ds(e0, CH_E)], ew_v.at[b], sls[b]))

        def fire_lin(k, b):
            for src, dst, sem in lin_refs(k, b):
                pltpu.async_copy(src, dst, sem)

        def wait_lin(k, b):
            for src, dst, sem in lin_refs(k, b):
                pltpu.make_async_copy(src, dst, sem).wait()

        def drain_scatters(b):
            for j in range(CH_R):
                pltpu.make_async_copy(sub(j), acc.at[col_v.at[b].at[j]],
                                      ss).wait()

        def phase(k, b):
            @pl.when(k < nk)
            def _():
                @pl.when(k > 0)
                def _():
                    drain_scatters(1 - b)
                wait_lin(k, b)
                gath = [pltpu.async_copy(h_hbm.at[row_v.at[b].at[j]],
                                         sub(j), sg)
                        for j in range(CH_R)]

                @pl.when(k + 1 < nk)
                def _():
                    fire_lin(k + 1, 1 - b)
                for j in range(CH_R):
                    gath[j].wait()

                    def scale_body(g, carry2, j=j):
                        o = pl.multiple_of(j * 128 + g * L, L)
                        ew16 = ew_v.at[b][pl.ds(o, L)]
                        for t in range(L):
                            ewb = lax.gather(
                                ew16, jnp.full((L, 1), t, jnp.int32), dnums,
                                (1,),
                                mode=lax.GatherScatterMode.PROMISE_IN_BOUNDS)
                            rows_v[o + t] = rows_v[o + t] * ewb
                        return carry2
                    lax.fori_loop(0, 128 // L, scale_body, 0)
                    pltpu.async_copy(sub(j), acc.at[col_v.at[b].at[j]],
                                     ss, add=True)

        fire_lin(0, 0)

        def pair_body(p, carry):
            phase(p * 2, 0)
            phase(p * 2 + 1, 1)
            return carry
        lax.fori_loop(0, (base_chunks + 2) // 2, pair_body, 0)

        drain_scatters(0)

        plsc.subcore_barrier()
        pltpu.sync_copy(acc.at[pl.ds(tbase, TSIZE)],
                        out_hbm.at[c, pl.ds(tbase, TSIZE)])

    return prop


def kernel(x, edge_index, edge_weight, degree, weight, bias):
    N, F = x.shape
    C = weight.shape[1]
    E = edge_index.shape[1]
    row = edge_index[0].reshape(E // 128, 128)
    col = edge_index[1].reshape(E // 128, 128)
    ew = edge_weight
    wpad = jnp.zeros((F, CPAD), jnp.float32).at[:, :C].set(weight)
    bpad = jnp.full((CPAD,), -1e30, jnp.float32).at[:C].set(bias).reshape(1, CPAD)
    deg = degree.reshape(N, 1)
    NP = N * CPAD // 128
    degp = jnp.repeat(degree, CPAD).reshape(NP, 128)
    grid = N // BN
    BNP = BN * CPAD // 128

    h0 = pl.pallas_call(
        _mm_body,
        grid=(grid,),
        in_specs=[pl.BlockSpec((BN, F), lambda i: (i, 0)),
                  pl.BlockSpec((F, CPAD), lambda i: (0, 0)),
                  pl.BlockSpec((BN, 1), lambda i: (i, 0))],
        out_specs=pl.BlockSpec((BN, CPAD), lambda i: (i, 0)),
        out_shape=jax.ShapeDtypeStruct((N, CPAD), jnp.float32),
    )(x, wpad, deg)

    prop = _make_prop(N, E)
    p1 = prop(h0, row, col, ew)

    h1p = pl.pallas_call(
        _combine_body,
        in_specs=[pl.BlockSpec((NC, NP, 128), lambda: (0, 0, 0)),
                  pl.BlockSpec((NP, 128), lambda: (0, 0))],
        out_specs=pl.BlockSpec((NP, 128), lambda: (0, 0)),
        out_shape=jax.ShapeDtypeStruct((NP, 128), jnp.float32),
    )(p1.reshape(NC, NP, 128), degp)

    p2 = prop(h1p.reshape(N, CPAD), row, col, ew)

    out = pl.pallas_call(
        _final_body,
        grid=(grid,),
        in_specs=[pl.BlockSpec((NC, BN, CPAD), lambda i: (0, i, 0)),
                  pl.BlockSpec((BN, 1), lambda i: (i, 0)),
                  pl.BlockSpec((1, CPAD), lambda i: (0, 0))],
        out_specs=pl.BlockSpec((BN, C), lambda i: (i, 0)),
        out_shape=jax.ShapeDtypeStruct((N, C), jnp.float32),
    )(p2, deg, bpad)

    return out
